# Initial kernel scaffold; baseline (speedup 1.0000x reference)
import jax
import jax.numpy as jnp
from jax.experimental import pallas as pl


def kernel(inputs, embeddings):
    raise NotImplementedError("write your pallas kernel here")


# speedup vs baseline: 7.9048x; 7.9048x over previous
import functools

import jax
import jax.numpy as jnp
from jax import lax
from jax.experimental import pallas as pl
from jax.experimental.pallas import tpu as pltpu
from jax.experimental.pallas import tpu_sc as plsc

MAXREL = 128
D = 64
V = 2 * MAXREL + 1
L = 2048
AROWS = 2 * L
BR = 8

_NW = 32
_RPW = AROWS // _NW


def _sc_build_a_body(emb_hbm, a_hbm, idx_v, rows_v, sem):
    wid = lax.axis_index("s") * 2 + lax.axis_index("c")
    base = wid * _RPW
    for c in range(_RPW // 16):
        t = lax.iota(jnp.int32, 16) + (base + c * 16)
        idx_v[pl.ds(c * 16, 16)] = jnp.clip((L - 1 + MAXREL) - t, 0, V - 1)
    pltpu.async_copy(emb_hbm.at[idx_v], rows_v, sem).wait()
    pltpu.sync_copy(rows_v, a_hbm.at[pl.ds(base, _RPW)])


def _sc_build_a(embeddings):
    embp = jnp.pad(embeddings, ((0, 0), (0, 128 - D)))
    mesh = plsc.VectorSubcoreMesh(core_axis_name="c", subcore_axis_name="s")
    return pl.kernel(
        _sc_build_a_body,
        mesh=mesh,
        out_type=jax.ShapeDtypeStruct((AROWS, 128), jnp.float32),
        scratch_types=[
            pltpu.VMEM((_RPW,), jnp.int32),
            pltpu.VMEM((_RPW, 128), jnp.float32),
            pltpu.SemaphoreType.DMA,
        ],
    )(embp)


def _tc_window_body(a_ref, o_ref):
    i0 = pl.program_id(0) * BR
    for r in range(BR):
        s = (L - 1) - (i0 + r)
        o_ref[r] = a_ref[pl.ds(s, L), 0:D]


def _tc_fill(a):
    return pl.pallas_call(
        _tc_window_body,
        grid=(L // BR,),
        in_specs=[pl.BlockSpec((AROWS, 128), lambda i: (0, 0))],
        out_specs=pl.BlockSpec((BR, L, D), lambda i: (i, 0, 0)),
        out_shape=jax.ShapeDtypeStruct((L, L, D), jnp.float32),
    )(a)


def kernel(inputs, embeddings):
    del inputs
    a = _sc_build_a(embeddings)
    return _tc_fill(a)


# speedup vs baseline: 8.1830x; 1.0352x over previous
import functools

import jax
import jax.numpy as jnp
from jax import lax
from jax.experimental import pallas as pl
from jax.experimental.pallas import tpu as pltpu
from jax.experimental.pallas import tpu_sc as plsc

MAXREL = 128
D = 64
V = 2 * MAXREL + 1
L = 2048
BROWS = 2 * L
BR = 8

_NW = 32
_RPW = BROWS // _NW


def _paired_table(embeddings):
    left = jnp.concatenate([embeddings, embeddings[V - 1:V]], axis=0)
    right = jnp.concatenate(
        [embeddings[0:1], embeddings[0:V - 1], embeddings[V - 1:V]], axis=0)
    return jnp.concatenate([left, right], axis=1)


def _sc_build_b_body(q_hbm, b_hbm, idx_v, rows_v, sem):
    wid = lax.axis_index("s") * 2 + lax.axis_index("c")
    base = wid * _RPW
    for c in range(_RPW // 16):
        t = lax.iota(jnp.int32, 16) + (base + c * 16)
        x = jnp.where(t < L, 2175 - 2 * t, 6270 - 2 * t)
        idx_v[pl.ds(c * 16, 16)] = jnp.clip(x, 0, V)
    pltpu.async_copy(q_hbm.at[idx_v], rows_v, sem).wait()
    pltpu.sync_copy(rows_v, b_hbm.at[pl.ds(base, _RPW)])


def _sc_build_b(embeddings):
    mesh = plsc.VectorSubcoreMesh(core_axis_name="c", subcore_axis_name="s")
    return pl.kernel(
        _sc_build_b_body,
        mesh=mesh,
        out_type=jax.ShapeDtypeStruct((BROWS, 128), jnp.float32),
        scratch_types=[
            pltpu.VMEM((_RPW,), jnp.int32),
            pltpu.VMEM((_RPW, 128), jnp.float32),
            pltpu.SemaphoreType.DMA,
        ],
    )(_paired_table(embeddings))


def _tc_window_body(b_ref, o_ref):
    i0 = pl.program_id(0) * BR
    for r in range(BR):
        i = i0 + r
        if r % 2 == 0:
            off = L + (2046 - i) // 2
        else:
            off = (2047 - i) // 2
        o_ref[r] = b_ref[pl.ds(off, L // 2), :]


def _tc_fill(b):
    return pl.pallas_call(
        _tc_window_body,
        grid=(L // BR,),
        in_specs=[pl.BlockSpec((BROWS, 128), lambda i: (0, 0))],
        out_specs=pl.BlockSpec((BR, L // 2, 128), lambda i: (i, 0, 0)),
        out_shape=jax.ShapeDtypeStruct((L, L // 2, 128), jnp.float32),
    )(b)


def kernel(inputs, embeddings):
    del inputs
    b = _sc_build_b(embeddings)
    return _tc_fill(b).reshape(L, L, D)


# speedup vs baseline: 8.1941x; 1.0013x over previous
import functools

import jax
import jax.numpy as jnp
from jax import lax
from jax.experimental import pallas as pl
from jax.experimental.pallas import tpu as pltpu
from jax.experimental.pallas import tpu_sc as plsc

MAXREL = 128
D = 64
V = 2 * MAXREL + 1
L = 2048
BROWS = 2 * L
BR = 8

_NW = 32
_RPW = BROWS // _NW


def _paired_table(embeddings):
    left = jnp.concatenate([embeddings, embeddings[V - 1:V]], axis=0)
    right = jnp.concatenate(
        [embeddings[0:1], embeddings[0:V - 1], embeddings[V - 1:V]], axis=0)
    return jnp.concatenate([left, right], axis=1)


def _sc_build_b_body(q_hbm, b_hbm, idx_v, rows_v, sem):
    wid = lax.axis_index("s") * 2 + lax.axis_index("c")
    base = wid * _RPW
    for c in range(_RPW // 16):
        t = lax.iota(jnp.int32, 16) + (base + c * 16)
        x = jnp.where(t < L, 2175 - 2 * t, 6270 - 2 * t)
        idx_v[pl.ds(c * 16, 16)] = jnp.clip(x, 0, V)
    pltpu.async_copy(q_hbm.at[idx_v], rows_v, sem).wait()
    pltpu.sync_copy(rows_v, b_hbm.at[pl.ds(base, _RPW)])


def _sc_build_b(embeddings):
    mesh = plsc.VectorSubcoreMesh(core_axis_name="c", subcore_axis_name="s")
    return pl.kernel(
        _sc_build_b_body,
        mesh=mesh,
        out_type=jax.ShapeDtypeStruct((BROWS, 128), jnp.float32),
        scratch_types=[
            pltpu.VMEM((_RPW,), jnp.int32),
            pltpu.VMEM((_RPW, 128), jnp.float32),
            pltpu.SemaphoreType.DMA,
        ],
    )(_paired_table(embeddings))


NBUF = 8


def _tc_window_body(b_ref, o_ref, sems):
    def row_copy(i, sem):
        off = jnp.where(i % 2 == 0, L + (2046 - i) // 2, (2047 - i) // 2)
        return pltpu.make_async_copy(
            b_ref.at[pl.ds(off, L // 2), :], o_ref.at[i], sem)

    def body(i, carry):
        @pl.when(i >= NBUF)
        def _():
            row_copy(i - NBUF, sems.at[i % NBUF]).wait()
        row_copy(i, sems.at[i % NBUF]).start()
        return carry

    lax.fori_loop(0, L, body, 0)

    def drain(i, carry):
        row_copy(i, sems.at[i % NBUF]).wait()
        return carry

    lax.fori_loop(L - NBUF, L, drain, 0)


def _tc_fill(b):
    return pl.pallas_call(
        _tc_window_body,
        in_specs=[pl.BlockSpec(memory_space=pltpu.VMEM)],
        out_specs=pl.BlockSpec(memory_space=pl.ANY),
        out_shape=jax.ShapeDtypeStruct((L, L // 2, 128), jnp.float32),
        scratch_shapes=[pltpu.SemaphoreType.DMA((NBUF,))],
    )(b)


def kernel(inputs, embeddings):
    del inputs
    b = _sc_build_b(embeddings)
    return _tc_fill(b).reshape(L, L, D)
